# baseline (speedup 1.0000x reference)
import functools

import jax
import jax.numpy as jnp
import numpy as np
from jax import lax
from jax.experimental import pallas as pl
from jax.experimental.pallas import tpu as pltpu
from jax.experimental.pallas import tpu_sc as plsc

NUM_FACTORS = 128
BATCH = 16384
NSPLIT = 2

_SC_INFO = plsc.get_sparse_core_info()
_NC = _SC_INFO.num_cores
_NS = _SC_INFO.num_subcores
_NW = _NC * _NS
_CHUNK = 64
_L = 16

_PACK_PERM = np.empty((NUM_FACTORS,), dtype=np.int32)
for _c in range(NUM_FACTORS // 32):
    for _k in range(16):
        _PACK_PERM[32 * _c + 2 * _k] = 32 * _c + _k
        _PACK_PERM[32 * _c + 2 * _k + 1] = 32 * _c + 16 + _k


def _make_sc_gather(batch):
    b_per_w = batch // _NW
    nchunks = b_per_w // _CHUNK

    def body(uid, iid, p_hbm, q_hbm, u_hbm, v_hbm,
             og, ou, ov,
             idx_u0, idx_i0, bp0, bq0, bu0, bv0, pg0, pu0, pv0,
             idx_u1, idx_i1, bp1, bq1, bu1, bv1, pg1, pu1, pv1,
             gsem0, gsem1, ssem0, ssem1):
        idx_u = (idx_u0, idx_u1)
        idx_i = (idx_i0, idx_i1)
        bp = (bp0, bp1)
        bq = (bq0, bq1)
        bu = (bu0, bu1)
        bv = (bv0, bv1)
        pg = (pg0, pg1)
        pu = (pu0, pu1)
        pv = (pv0, pv1)
        gsem = (gsem0, gsem1)
        ssem = (ssem0, ssem1)

        wid = lax.axis_index("s") * _NC + lax.axis_index("c")
        base = wid * b_per_w

        def issue_gathers(c, d):
            row0 = base + c * _CHUNK
            pltpu.sync_copy(uid.at[pl.ds(row0, _CHUNK)], idx_u[d])
            pltpu.sync_copy(iid.at[pl.ds(row0, _CHUNK)], idx_i[d])
            return (
                pltpu.async_copy(p_hbm.at[idx_u[d]], bp[d], gsem[d]),
                pltpu.async_copy(q_hbm.at[idx_i[d]], bq[d], gsem[d]),
                pltpu.async_copy(u_hbm.at[idx_u[d]], bu[d], gsem[d]),
                pltpu.async_copy(v_hbm.at[idx_i[d]], bv[d], gsem[d]),
            )

        def issue_scatters(c, d):
            row0 = base + c * _CHUNK
            return (
                pltpu.async_copy(pg[d], og.at[pl.ds(row0, _CHUNK)], ssem[d]),
                pltpu.async_copy(pu[d], ou.at[pl.ds(row0, _CHUNK)], ssem[d]),
                pltpu.async_copy(pv[d], ov.at[pl.ds(row0, _CHUNK)], ssem[d]),
            )

        def pack_rows(d):
            bp_d, bq_d, bu_d, bv_d = bp[d], bq[d], bu[d], bv[d]
            pg_d, pu_d, pv_d = pg[d], pu[d], pv[d]
            half = jnp.int32(0x8000)
            himask = jnp.int32(-65536)

            def pack2(a, b):
                ia = lax.bitcast_convert_type(a, jnp.int32)
                ib = lax.bitcast_convert_type(b, jnp.int32)
                lo = lax.shift_right_logical(ia + half, jnp.int32(16))
                hi = (ib + half) & himask
                return lo | hi

            @plsc.parallel_loop(0, _CHUNK, unroll=4)
            def row_body(r):
                for j in range(NUM_FACTORS // 32):
                    lo = pl.ds(32 * j, _L)
                    hi = pl.ds(32 * j + _L, _L)
                    dst = pl.ds(_L * j, _L)
                    ga = bp_d[r, lo] * bq_d[r, lo]
                    gb = bp_d[r, hi] * bq_d[r, hi]
                    pg_d[r, dst] = pack2(ga, gb)
                    pu_d[r, dst] = pack2(bu_d[r, lo], bu_d[r, hi])
                    pv_d[r, dst] = pack2(bv_d[r, lo], bv_d[r, hi])

        gcopies = [None, None]
        scopies = [None, None]
        gcopies[0] = issue_gathers(0, 0)
        if nchunks > 1:
            gcopies[1] = issue_gathers(1, 1)
        for c in range(nchunks):
            d = c % 2
            for cp in gcopies[d]:
                cp.wait()
            if c >= 2:
                for cp in scopies[d]:
                    cp.wait()
            pack_rows(d)
            scopies[d] = issue_scatters(c, d)
            if c + 2 < nchunks:
                gcopies[d] = issue_gathers(c + 2, d)
        for d in range(min(2, nchunks)):
            for cp in scopies[d]:
                cp.wait()

    out_shape = jax.ShapeDtypeStruct((batch, NUM_FACTORS // 2), jnp.int32)

    def stage_bufs():
        return [
            pltpu.VMEM((_CHUNK,), jnp.int32),
            pltpu.VMEM((_CHUNK,), jnp.int32),
            pltpu.VMEM((_CHUNK, NUM_FACTORS), jnp.float32),
            pltpu.VMEM((_CHUNK, NUM_FACTORS), jnp.float32),
            pltpu.VMEM((_CHUNK, NUM_FACTORS), jnp.float32),
            pltpu.VMEM((_CHUNK, NUM_FACTORS), jnp.float32),
            pltpu.VMEM((_CHUNK, NUM_FACTORS // 2), jnp.int32),
            pltpu.VMEM((_CHUNK, NUM_FACTORS // 2), jnp.int32),
            pltpu.VMEM((_CHUNK, NUM_FACTORS // 2), jnp.int32),
        ]

    return functools.partial(
        pl.kernel,
        mesh=plsc.VectorSubcoreMesh(core_axis_name="c", subcore_axis_name="s"),
        out_type=(out_shape, out_shape, out_shape),
        scratch_types=stage_bufs() + stage_bufs() + [
            pltpu.SemaphoreType.DMA,
            pltpu.SemaphoreType.DMA,
            pltpu.SemaphoreType.DMA,
            pltpu.SemaphoreType.DMA,
        ],
    )(body)


_R = 4096


def _tc_mlp_body(gm, um, vm, w1u, w1v, b1, w2, b2, w3, b3, wog, woh, out):
    h1 = jnp.dot(um[...], w1u[...], preferred_element_type=jnp.float32)
    h1 += jnp.dot(vm[...], w1v[...], preferred_element_type=jnp.float32)
    h1 = jnp.maximum(h1 + b1[...][None, :], 0.0).astype(jnp.bfloat16)
    h2 = jnp.dot(h1, w2[...], preferred_element_type=jnp.float32)
    h2 = jnp.maximum(h2 + b2[...][None, :], 0.0).astype(jnp.bfloat16)
    h3 = jnp.dot(h2, w3[...], preferred_element_type=jnp.float32)
    h3 = jnp.maximum(h3 + b3[...][None, :], 0.0)
    z = jnp.dot(gm[...], wog[...], preferred_element_type=jnp.float32)
    z += jnp.dot(h3, woh[...], preferred_element_type=jnp.float32)
    out[...] = jax.nn.sigmoid(z)


def _tc_mlp(gm, um, vm, w1u, w1v, b1, w2, b2, w3, b3, wog, woh):
    batch = gm.shape[0]
    grid = (batch // _R,)
    row_spec = pl.BlockSpec((_R, NUM_FACTORS), lambda i: (i, 0))
    full = lambda s: pl.BlockSpec(s, lambda i: (0,) * len(s))
    return pl.pallas_call(
        _tc_mlp_body,
        grid=grid,
        in_specs=[
            row_spec, row_spec, row_spec,
            full(w1u.shape), full(w1v.shape), full(b1.shape),
            full(w2.shape), full(b2.shape),
            full(w3.shape), full(b3.shape),
            full(wog.shape), full(woh.shape),
        ],
        out_specs=pl.BlockSpec((_R, 1), lambda i: (i, 0)),
        out_shape=jax.ShapeDtypeStruct((batch, 1), jnp.float32),
    )(gm, um, vm, w1u, w1v, b1, w2, b2, w3, b3, wog, woh)


def kernel(user_ids, item_ids, P, Q, U, V, W1, b1, W2, b2, W3, b3, Wo):
    perm = jnp.asarray(_PACK_PERM)
    bf = jnp.bfloat16
    w1u = W1[:NUM_FACTORS][perm].astype(bf)
    w1v = W1[NUM_FACTORS:][perm].astype(bf)
    wog = Wo[:NUM_FACTORS][perm].astype(bf)
    woh = Wo[NUM_FACTORS:]

    half = BATCH // NSPLIT
    sc = _make_sc_gather(half)

    def unpack_bf16(x):
        y = lax.bitcast_convert_type(x, bf)
        return y.reshape(x.shape[0], NUM_FACTORS)

    gathered = []
    for s in range(NSPLIT):
        sl = slice(s * half, (s + 1) * half)
        gathered.append(sc(user_ids[sl], item_ids[sl], P, Q, U, V))
    outs = [
        _tc_mlp(unpack_bf16(gm), unpack_bf16(um), unpack_bf16(vm),
                w1u, w1v, b1,
                W2.astype(bf), b2, W3.astype(bf), b3, wog, woh)
        for (gm, um, vm) in gathered
    ]
    return jnp.concatenate(outs, axis=0)


# speedup vs baseline: 3.0791x; 3.0791x over previous
import functools

import jax
import jax.numpy as jnp
from jax import lax
from jax.experimental import pallas as pl
from jax.experimental.pallas import tpu as pltpu
from jax.experimental.pallas import tpu_sc as plsc

NUM_FACTORS = 128
BATCH = 16384
NSPLIT = 2

_SC_INFO = plsc.get_sparse_core_info()
_NC = _SC_INFO.num_cores
_NS = _SC_INFO.num_subcores
_NW = _NC * _NS
_CHUNK = 64


def _make_sc_gather(batch, row_offset):
    b_per_w = batch // _NW
    nchunks = b_per_w // _CHUNK

    def body(uid, iid, p_hbm, q_hbm, u_hbm, v_hbm,
             op, oq, ou, ov,
             idx_u0, idx_i0, bp0, bq0, bu0, bv0,
             idx_u1, idx_i1, bp1, bq1, bu1, bv1,
             gsem0, gsem1, ssem0, ssem1):
        idx_u = (idx_u0, idx_u1)
        idx_i = (idx_i0, idx_i1)
        bp = (bp0, bp1)
        bq = (bq0, bq1)
        bu = (bu0, bu1)
        bv = (bv0, bv1)
        gsem = (gsem0, gsem1)
        ssem = (ssem0, ssem1)

        wid = lax.axis_index("s") * _NC + lax.axis_index("c")
        base = wid * b_per_w
        ibase = row_offset + base

        def issue_gathers(c, d):
            row0 = base + c * _CHUNK
            irow0 = ibase + c * _CHUNK
            pltpu.sync_copy(uid.at[pl.ds(irow0, _CHUNK)], idx_u[d])
            pltpu.sync_copy(iid.at[pl.ds(irow0, _CHUNK)], idx_i[d])
            return (
                pltpu.async_copy(p_hbm.at[idx_u[d]], bp[d], gsem[d]),
                pltpu.async_copy(q_hbm.at[idx_i[d]], bq[d], gsem[d]),
                pltpu.async_copy(u_hbm.at[idx_u[d]], bu[d], gsem[d]),
                pltpu.async_copy(v_hbm.at[idx_i[d]], bv[d], gsem[d]),
            )

        def issue_scatters(c, d):
            row0 = base + c * _CHUNK
            return (
                pltpu.async_copy(bp[d], op.at[pl.ds(row0, _CHUNK)], ssem[d]),
                pltpu.async_copy(bq[d], oq.at[pl.ds(row0, _CHUNK)], ssem[d]),
                pltpu.async_copy(bu[d], ou.at[pl.ds(row0, _CHUNK)], ssem[d]),
                pltpu.async_copy(bv[d], ov.at[pl.ds(row0, _CHUNK)], ssem[d]),
            )

        gcopies = [None, None]
        scopies = [None, None]
        gcopies[0] = issue_gathers(0, 0)
        if nchunks > 1:
            gcopies[1] = issue_gathers(1, 1)
        for c in range(nchunks):
            d = c % 2
            for cp in gcopies[d]:
                cp.wait()
            scopies[d] = issue_scatters(c, d)
            if c + 2 < nchunks:
                for cp in scopies[d]:
                    cp.wait()
                gcopies[d] = issue_gathers(c + 2, d)
        for d in range(min(2, nchunks)):
            for cp in scopies[d]:
                cp.wait()

    row_shape = jax.ShapeDtypeStruct((batch, NUM_FACTORS), jnp.float32)

    def row_bufs():
        return [
            pltpu.VMEM((_CHUNK,), jnp.int32),
            pltpu.VMEM((_CHUNK,), jnp.int32),
            pltpu.VMEM((_CHUNK, NUM_FACTORS), jnp.float32),
            pltpu.VMEM((_CHUNK, NUM_FACTORS), jnp.float32),
            pltpu.VMEM((_CHUNK, NUM_FACTORS), jnp.float32),
            pltpu.VMEM((_CHUNK, NUM_FACTORS), jnp.float32),
        ]

    return functools.partial(
        pl.kernel,
        mesh=plsc.VectorSubcoreMesh(core_axis_name="c", subcore_axis_name="s"),
        out_type=(row_shape, row_shape, row_shape, row_shape),
        scratch_types=row_bufs() + row_bufs() + [
            pltpu.SemaphoreType.DMA,
            pltpu.SemaphoreType.DMA,
            pltpu.SemaphoreType.DMA,
            pltpu.SemaphoreType.DMA,
        ],
    )(body)


_R = 4096


def _tc_mlp_body(pm, qm, um, vm, w1, b1, w2, b2, w3, b3, wo, out):
    bf = jnp.bfloat16
    xu = um[...].astype(bf)
    xv = vm[...].astype(bf)
    w1v = w1[...].astype(bf)
    h1 = jnp.dot(xu, w1v[:NUM_FACTORS], preferred_element_type=jnp.float32)
    h1 += jnp.dot(xv, w1v[NUM_FACTORS:], preferred_element_type=jnp.float32)
    h1 = jnp.maximum(h1 + b1[...][None, :], 0.0).astype(bf)
    h2 = jnp.dot(h1, w2[...].astype(bf), preferred_element_type=jnp.float32)
    h2 = jnp.maximum(h2 + b2[...][None, :], 0.0).astype(bf)
    h3 = jnp.dot(h2, w3[...].astype(bf), preferred_element_type=jnp.float32)
    h3 = jnp.maximum(h3 + b3[...][None, :], 0.0)
    gmf = pm[...] * qm[...]
    wov = wo[...]
    z = jnp.dot(gmf, wov[:NUM_FACTORS], preferred_element_type=jnp.float32)
    z += jnp.dot(h3, wov[NUM_FACTORS:], preferred_element_type=jnp.float32)
    out[...] = jax.nn.sigmoid(z)


def _tc_mlp(pm, qm, um, vm, w1, b1, w2, b2, w3, b3, wo):
    batch = pm.shape[0]
    grid = (batch // _R,)
    row_spec = pl.BlockSpec((_R, NUM_FACTORS), lambda i: (i, 0))
    full = lambda s: pl.BlockSpec(s, lambda i: (0,) * len(s))
    return pl.pallas_call(
        _tc_mlp_body,
        grid=grid,
        in_specs=[
            row_spec, row_spec, row_spec, row_spec,
            full(w1.shape), full(b1.shape),
            full(w2.shape), full(b2.shape),
            full(w3.shape), full(b3.shape),
            full(wo.shape),
        ],
        out_specs=pl.BlockSpec((_R, 1), lambda i: (i, 0)),
        out_shape=jax.ShapeDtypeStruct((batch, 1), jnp.float32),
    )(pm, qm, um, vm, w1, b1, w2, b2, w3, b3, wo)


def kernel(user_ids, item_ids, P, Q, U, V, W1, b1, W2, b2, W3, b3, Wo):
    half = BATCH // NSPLIT
    gathered = []
    for s in range(NSPLIT):
        sc = _make_sc_gather(half, s * half)
        gathered.append(sc(user_ids, item_ids, P, Q, U, V))
    outs = [
        _tc_mlp(pm, qm, um, vm, W1, b1, W2, b2, W3, b3, Wo)
        for (pm, qm, um, vm) in gathered
    ]
    return jnp.concatenate(outs, axis=0)
